# Initial kernel scaffold; baseline (speedup 1.0000x reference)
import jax
import jax.numpy as jnp
from jax.experimental import pallas as pl


def kernel(x, edge_index, Wl0, Wr0, Wl1, Wr1, Wl2, Wr2, g0, b0, g1, b1, g2, b2, W1, bb1, gm, bm, W2, bb2):
    raise NotImplementedError("write your pallas kernel here")


# speedup vs baseline: 2.4306x; 2.4306x over previous
import functools

import jax
import jax.numpy as jnp
from jax import lax
from jax.experimental import pallas as pl
from jax.experimental.pallas import tpu as pltpu
from jax.experimental.pallas import tpu_sc as plsc

NN = 10000
EE = 320000
DD = 128
CC = 64

NC, NS = 2, 16
NW = NC * NS
CHUNK = 128
KCH = 80
EPT = KCH * CHUNK
EPAD = EPT * NW
NACC = 10240
STRIPE = NACC // NS
DUMMY = 10016
CW = 16


@functools.lru_cache(maxsize=None)
def _get_sc_agg():
  mesh = plsc.VectorSubcoreMesh(core_axis_name="c", subcore_axis_name="s")

  @functools.partial(
      pl.kernel, mesh=mesh,
      out_type=jax.ShapeDtypeStruct((NC, NACC, DD), jnp.float32),
      scratch_types=[
          pltpu.VMEM((KCH, CHUNK), jnp.int32),
          pltpu.VMEM((KCH, CHUNK), jnp.int32),
          pltpu.VMEM((CHUNK, DD), jnp.float32),
          pltpu.VMEM_SHARED((NACC, DD), jnp.float32),
          pltpu.SemaphoreType.DMA,
      ],
  )
  def _sc_agg(h_hbm, src_hbm, dst_hbm, zstripe_hbm,
              out_f, src_v, dst_v, rows_v, accf, sem):
    c = lax.axis_index("c")
    s = lax.axis_index("s")
    wid = c * NS + s

    pltpu.sync_copy(zstripe_hbm, accf.at[pl.ds(s * STRIPE, STRIPE)])

    pltpu.sync_copy(src_hbm.at[pl.ds(wid * KCH, KCH)], src_v)
    pltpu.sync_copy(dst_hbm.at[pl.ds(wid * KCH, KCH)], dst_v)
    plsc.subcore_barrier()

    def step(j, carry):
        pltpu.async_copy(h_hbm.at[src_v.at[j]], rows_v, sem).wait()
        pltpu.sync_copy(rows_v, accf.at[dst_v.at[j]], add=True)
        return carry
    lax.fori_loop(0, KCH, step, 0)

    plsc.subcore_barrier()
    pltpu.sync_copy(accf.at[pl.ds(s * STRIPE, STRIPE)],
                    out_f.at[c, pl.ds(s * STRIPE, STRIPE)])

  return _sc_agg


@functools.lru_cache(maxsize=None)
def _get_sc_cnt():
  mesh = plsc.VectorSubcoreMesh(core_axis_name="c", subcore_axis_name="s")

  @functools.partial(
      pl.kernel, mesh=mesh,
      out_type=jax.ShapeDtypeStruct((NC, NACC, DD), jnp.float32),
      scratch_types=[
          pltpu.VMEM((KCH, CHUNK), jnp.int32),
          pltpu.VMEM((CHUNK, DD), jnp.float32),
          pltpu.VMEM_SHARED((NACC, DD), jnp.float32),
      ],
  )
  def _sc_cnt(dst_hbm, zstripe_hbm, ones_hbm, out_c, dst_v, ones_v, accc):
    c = lax.axis_index("c")
    s = lax.axis_index("s")
    wid = c * NS + s

    pltpu.sync_copy(zstripe_hbm, accc.at[pl.ds(s * STRIPE, STRIPE)])
    pltpu.sync_copy(ones_hbm, ones_v)
    pltpu.sync_copy(dst_hbm.at[pl.ds(wid * KCH, KCH)], dst_v)
    plsc.subcore_barrier()

    def step(j, carry):
        pltpu.sync_copy(ones_v, accc.at[dst_v.at[j]], add=True)
        return carry
    lax.fori_loop(0, KCH, step, 0)

    plsc.subcore_barrier()
    pltpu.sync_copy(accc.at[pl.ds(s * STRIPE, STRIPE)],
                    out_c.at[c, pl.ds(s * STRIPE, STRIPE)])

  return _sc_cnt


RB = 1000


def _conv_body(h_ref, pf_ref, pc_ref, wl_ref, wr_ref, y_ref, st_ref, acc_ref):
    i = pl.program_id(0)

    @pl.when(i == 0)
    def _init():
        acc_ref[...] = jnp.zeros_like(acc_ref)

    psum = pf_ref[0] + pf_ref[1]
    cnt = pc_ref[0, :, 0:1] + pc_ref[1, :, 0:1]
    inv = 1.0 / jnp.maximum(cnt, 1.0)
    m = jnp.dot(psum, wl_ref[...], preferred_element_type=jnp.float32) * inv
    y = m + jnp.dot(h_ref[...], wr_ref[...], preferred_element_type=jnp.float32)
    y_ref[...] = y
    acc_ref[0:1, :] += jnp.sum(y, axis=0, keepdims=True)
    acc_ref[1:2, :] += jnp.sum(y * y, axis=0, keepdims=True)

    @pl.when(i == pl.num_programs(0) - 1)
    def _fin():
        st_ref[...] = acc_ref[...]


def _conv(h, pf, pc, wl, wr):
    return pl.pallas_call(
        _conv_body,
        grid=(NN // RB,),
        in_specs=[
            pl.BlockSpec((RB, DD), lambda i: (i, 0)),
            pl.BlockSpec((NC, RB, DD), lambda i: (0, i, 0)),
            pl.BlockSpec((NC, RB, DD), lambda i: (0, i, 0)),
            pl.BlockSpec((DD, DD), lambda i: (0, 0)),
            pl.BlockSpec((DD, DD), lambda i: (0, 0)),
        ],
        out_specs=[
            pl.BlockSpec((RB, DD), lambda i: (i, 0)),
            pl.BlockSpec((2, DD), lambda i: (0, 0)),
        ],
        out_shape=[
            jax.ShapeDtypeStruct((NN, DD), jnp.float32),
            jax.ShapeDtypeStruct((2, DD), jnp.float32),
        ],
        scratch_shapes=[pltpu.VMEM((2, DD), jnp.float32)],
    )(h, pf, pc, wl, wr)


def _bnrelu_body(y_ref, st_ref, g_ref, b_ref, o_ref):
    mu = st_ref[0:1, :] / NN
    var = st_ref[1:2, :] / NN - mu * mu
    scale = g_ref[...] * lax.rsqrt(var + 1e-5)
    o_ref[...] = jnp.maximum((y_ref[...] - mu) * scale + b_ref[...], 0.0)


def _bnrelu(y, st, g, b):
    return pl.pallas_call(
        _bnrelu_body,
        grid=(NN // RB,),
        in_specs=[
            pl.BlockSpec((RB, DD), lambda i: (i, 0)),
            pl.BlockSpec((2, DD), lambda i: (0, 0)),
            pl.BlockSpec((1, DD), lambda i: (0, 0)),
            pl.BlockSpec((1, DD), lambda i: (0, 0)),
        ],
        out_specs=pl.BlockSpec((RB, DD), lambda i: (i, 0)),
        out_shape=jax.ShapeDtypeStruct((NN, DD), jnp.float32),
    )(y, st, g.reshape(1, DD), b.reshape(1, DD))


def _mlp1_body(x_ref, h1_ref, h2_ref, h3_ref, w1_ref, bb1_ref,
               z_ref, st_ref, acc_ref):
    i = pl.program_id(0)

    @pl.when(i == 0)
    def _init():
        acc_ref[...] = jnp.zeros_like(acc_ref)

    z = (jnp.dot(x_ref[...], w1_ref[0 * DD:1 * DD], preferred_element_type=jnp.float32)
         + jnp.dot(h1_ref[...], w1_ref[1 * DD:2 * DD], preferred_element_type=jnp.float32)
         + jnp.dot(h2_ref[...], w1_ref[2 * DD:3 * DD], preferred_element_type=jnp.float32)
         + jnp.dot(h3_ref[...], w1_ref[3 * DD:4 * DD], preferred_element_type=jnp.float32)
         + bb1_ref[...])
    z_ref[...] = z
    acc_ref[0:1, :] += jnp.sum(z, axis=0, keepdims=True)
    acc_ref[1:2, :] += jnp.sum(z * z, axis=0, keepdims=True)

    @pl.when(i == pl.num_programs(0) - 1)
    def _fin():
        st_ref[...] = acc_ref[...]


def _mlp1(x, h1, h2, h3, w1, bb1):
    return pl.pallas_call(
        _mlp1_body,
        grid=(NN // RB,),
        in_specs=[
            pl.BlockSpec((RB, DD), lambda i: (i, 0)),
            pl.BlockSpec((RB, DD), lambda i: (i, 0)),
            pl.BlockSpec((RB, DD), lambda i: (i, 0)),
            pl.BlockSpec((RB, DD), lambda i: (i, 0)),
            pl.BlockSpec((4 * DD, 2 * CC), lambda i: (0, 0)),
            pl.BlockSpec((1, 2 * CC), lambda i: (0, 0)),
        ],
        out_specs=[
            pl.BlockSpec((RB, 2 * CC), lambda i: (i, 0)),
            pl.BlockSpec((2, 2 * CC), lambda i: (0, 0)),
        ],
        out_shape=[
            jax.ShapeDtypeStruct((NN, 2 * CC), jnp.float32),
            jax.ShapeDtypeStruct((2, 2 * CC), jnp.float32),
        ],
        scratch_shapes=[pltpu.VMEM((2, 2 * CC), jnp.float32)],
    )(x, h1, h2, h3, w1, bb1.reshape(1, 2 * CC))


def _mlp2_body(z_ref, st_ref, gm_ref, bm_ref, w2_ref, bb2_ref, o_ref):
    mu = st_ref[0:1, :] / NN
    var = st_ref[1:2, :] / NN - mu * mu
    scale = gm_ref[...] * lax.rsqrt(var + 1e-5)
    t = (z_ref[...] - mu) * scale + bm_ref[...]
    t = jnp.where(t >= 0.0, t, 0.01 * t)
    o_ref[...] = jnp.dot(t, w2_ref[...], preferred_element_type=jnp.float32) + bb2_ref[...]


def _mlp2(z, st, gm, bm, w2, bb2):
    return pl.pallas_call(
        _mlp2_body,
        grid=(NN // RB,),
        in_specs=[
            pl.BlockSpec((RB, 2 * CC), lambda i: (i, 0)),
            pl.BlockSpec((2, 2 * CC), lambda i: (0, 0)),
            pl.BlockSpec((1, 2 * CC), lambda i: (0, 0)),
            pl.BlockSpec((1, 2 * CC), lambda i: (0, 0)),
            pl.BlockSpec((2 * CC, CC), lambda i: (0, 0)),
            pl.BlockSpec((1, CC), lambda i: (0, 0)),
        ],
        out_specs=pl.BlockSpec((RB, CC), lambda i: (i, 0)),
        out_shape=jax.ShapeDtypeStruct((NN, CC), jnp.float32),
    )(z, st, gm.reshape(1, 2 * CC), bm.reshape(1, 2 * CC), w2, bb2.reshape(1, CC))


def kernel(x, edge_index, Wl0, Wr0, Wl1, Wr1, Wl2, Wr2,
           g0, b0, g1, b1, g2, b2, W1, bb1, gm, bm, W2, bb2):
    src = edge_index[0]
    dst = edge_index[1]
    pad = EPAD - EE
    src2d = jnp.concatenate(
        [src, jnp.zeros((pad,), jnp.int32)]).reshape(EPAD // CHUNK, CHUNK)
    pad_dst = NN + (jnp.arange(pad, dtype=jnp.int32) % (NACC - NN))
    dst2d = jnp.concatenate([dst, pad_dst]).reshape(EPAD // CHUNK, CHUNK)
    zstripe = jnp.zeros((STRIPE, DD), jnp.float32)
    ones = jnp.ones((CHUNK, DD), jnp.float32)

    sc_cnt = _get_sc_cnt()
    sc_agg = _get_sc_agg()
    pc = sc_cnt(dst2d, zstripe, ones)
    pf0 = sc_agg(x, src2d, dst2d, zstripe)
    y1, st1 = _conv(x, pf0, pc, Wl0, Wr0)
    h1 = _bnrelu(y1, st1, g0, b0)

    pf1 = sc_agg(h1, src2d, dst2d, zstripe)
    y2, st2 = _conv(h1, pf1, pc, Wl1, Wr1)
    h2 = _bnrelu(y2, st2, g1, b1)

    pf2 = sc_agg(h2, src2d, dst2d, zstripe)
    y3, st3 = _conv(h2, pf2, pc, Wl2, Wr2)
    h3 = _bnrelu(y3, st3, g2, b2)

    z, stz = _mlp1(x, h1, h2, h3, W1, bb1)
    return _mlp2(z, stz, gm, bm, W2, bb2)


# speedup vs baseline: 6.6272x; 2.7265x over previous
import functools

import jax
import jax.numpy as jnp
from jax import lax
from jax.experimental import pallas as pl
from jax.experimental.pallas import tpu as pltpu
from jax.experimental.pallas import tpu_sc as plsc

NN = 10000
EE = 320000
DD = 128
CC = 64

NC, NS = 2, 16
NW = NC * NS
CHUNK = 128
KCH = 80
EPT = KCH * CHUNK
EPAD = EPT * NW
NACC = 10240
STRIPE = NACC // NS
DUMMY = 10016
CW = 16


@functools.lru_cache(maxsize=None)
def _get_sc_agg():
  mesh = plsc.VectorSubcoreMesh(core_axis_name="c", subcore_axis_name="s")

  @functools.partial(
      pl.kernel, mesh=mesh,
      out_type=jax.ShapeDtypeStruct((NC, NACC, DD), jnp.float32),
      scratch_types=[
          pltpu.VMEM((KCH, CHUNK), jnp.int32),
          pltpu.VMEM((KCH, CHUNK), jnp.int32),
          pltpu.VMEM((CHUNK, DD), jnp.float32),
          pltpu.VMEM_SHARED((NACC, DD), jnp.float32),
          pltpu.SemaphoreType.DMA,
      ],
  )
  def _sc_agg(h_hbm, src_hbm, dst_hbm, zstripe_hbm,
              out_f, src_v, dst_v, rows_v, accf, sem):
    c = lax.axis_index("c")
    s = lax.axis_index("s")
    wid = c * NS + s

    pltpu.sync_copy(zstripe_hbm, accf.at[pl.ds(s * STRIPE, STRIPE)])

    pltpu.sync_copy(src_hbm.at[pl.ds(wid * KCH, KCH)], src_v)
    pltpu.sync_copy(dst_hbm.at[pl.ds(wid * KCH, KCH)], dst_v)
    plsc.subcore_barrier()

    def step(j, carry):
        pltpu.async_copy(h_hbm.at[src_v.at[j]], rows_v, sem).wait()
        pltpu.sync_copy(rows_v, accf.at[dst_v.at[j]], add=True)
        return carry
    lax.fori_loop(0, KCH, step, 0)

    plsc.subcore_barrier()
    pltpu.sync_copy(accf.at[pl.ds(s * STRIPE, STRIPE)],
                    out_f.at[c, pl.ds(s * STRIPE, STRIPE)])

  return _sc_agg


@functools.lru_cache(maxsize=None)
def _get_sc_cnt():
  mesh = plsc.VectorSubcoreMesh(core_axis_name="c", subcore_axis_name="s")

  @functools.partial(
      pl.kernel, mesh=mesh,
      out_type=jax.ShapeDtypeStruct((NC, NACC, DD), jnp.float32),
      scratch_types=[
          pltpu.VMEM((KCH, CHUNK), jnp.int32),
          pltpu.VMEM((CHUNK, DD), jnp.float32),
          pltpu.VMEM_SHARED((NACC, DD), jnp.float32),
      ],
  )
  def _sc_cnt(dst_hbm, zstripe_hbm, ones_hbm, out_c, dst_v, ones_v, accc):
    c = lax.axis_index("c")
    s = lax.axis_index("s")
    wid = c * NS + s

    pltpu.sync_copy(zstripe_hbm, accc.at[pl.ds(s * STRIPE, STRIPE)])
    pltpu.sync_copy(ones_hbm, ones_v)
    pltpu.sync_copy(dst_hbm.at[pl.ds(wid * KCH, KCH)], dst_v)
    plsc.subcore_barrier()

    def step(j, carry):
        pltpu.sync_copy(ones_v, accc.at[dst_v.at[j]], add=True)
        return carry
    lax.fori_loop(0, KCH, step, 0)

    plsc.subcore_barrier()
    pltpu.sync_copy(accc.at[pl.ds(s * STRIPE, STRIPE)],
                    out_c.at[c, pl.ds(s * STRIPE, STRIPE)])

  return _sc_cnt


RB = 1000


def _conv_body(h_ref, pf_ref, pc_ref, wl_ref, wr_ref, y_ref, st_ref, acc_ref):
    i = pl.program_id(0)

    @pl.when(i == 0)
    def _init():
        acc_ref[...] = jnp.zeros_like(acc_ref)

    psum = pf_ref[0] + pf_ref[1]
    cnt = pc_ref[0, :, 0:1] + pc_ref[1, :, 0:1]
    inv = 1.0 / jnp.maximum(cnt, 1.0)
    m = jnp.dot(psum, wl_ref[...], preferred_element_type=jnp.float32) * inv
    y = m + jnp.dot(h_ref[...], wr_ref[...], preferred_element_type=jnp.float32)
    y_ref[...] = y
    acc_ref[0:1, :] += jnp.sum(y, axis=0, keepdims=True)
    acc_ref[1:2, :] += jnp.sum(y * y, axis=0, keepdims=True)

    @pl.when(i == pl.num_programs(0) - 1)
    def _fin():
        st_ref[...] = acc_ref[...]


def _conv(h, pf, pc, wl, wr):
    return pl.pallas_call(
        _conv_body,
        grid=(NN // RB,),
        in_specs=[
            pl.BlockSpec((RB, DD), lambda i: (i, 0)),
            pl.BlockSpec((NC, RB, DD), lambda i: (0, i, 0)),
            pl.BlockSpec((NC, RB, DD), lambda i: (0, i, 0)),
            pl.BlockSpec((DD, DD), lambda i: (0, 0)),
            pl.BlockSpec((DD, DD), lambda i: (0, 0)),
        ],
        out_specs=[
            pl.BlockSpec((RB, DD), lambda i: (i, 0)),
            pl.BlockSpec((2, DD), lambda i: (0, 0)),
        ],
        out_shape=[
            jax.ShapeDtypeStruct((NN, DD), jnp.float32),
            jax.ShapeDtypeStruct((2, DD), jnp.float32),
        ],
        scratch_shapes=[pltpu.VMEM((2, DD), jnp.float32)],
    )(h, pf, pc, wl, wr)


def _bnrelu_body(y_ref, st_ref, g_ref, b_ref, o_ref):
    mu = st_ref[0:1, :] / NN
    var = st_ref[1:2, :] / NN - mu * mu
    scale = g_ref[...] * lax.rsqrt(var + 1e-5)
    o_ref[...] = jnp.maximum((y_ref[...] - mu) * scale + b_ref[...], 0.0)


def _bnrelu(y, st, g, b):
    return pl.pallas_call(
        _bnrelu_body,
        grid=(NN // RB,),
        in_specs=[
            pl.BlockSpec((RB, DD), lambda i: (i, 0)),
            pl.BlockSpec((2, DD), lambda i: (0, 0)),
            pl.BlockSpec((1, DD), lambda i: (0, 0)),
            pl.BlockSpec((1, DD), lambda i: (0, 0)),
        ],
        out_specs=pl.BlockSpec((RB, DD), lambda i: (i, 0)),
        out_shape=jax.ShapeDtypeStruct((NN, DD), jnp.float32),
    )(y, st, g.reshape(1, DD), b.reshape(1, DD))


def _mlp1_body(x_ref, h1_ref, h2_ref, h3_ref, w1_ref, bb1_ref,
               z_ref, st_ref, acc_ref):
    i = pl.program_id(0)

    @pl.when(i == 0)
    def _init():
        acc_ref[...] = jnp.zeros_like(acc_ref)

    z = (jnp.dot(x_ref[...], w1_ref[0 * DD:1 * DD], preferred_element_type=jnp.float32)
         + jnp.dot(h1_ref[...], w1_ref[1 * DD:2 * DD], preferred_element_type=jnp.float32)
         + jnp.dot(h2_ref[...], w1_ref[2 * DD:3 * DD], preferred_element_type=jnp.float32)
         + jnp.dot(h3_ref[...], w1_ref[3 * DD:4 * DD], preferred_element_type=jnp.float32)
         + bb1_ref[...])
    z_ref[...] = z
    acc_ref[0:1, :] += jnp.sum(z, axis=0, keepdims=True)
    acc_ref[1:2, :] += jnp.sum(z * z, axis=0, keepdims=True)

    @pl.when(i == pl.num_programs(0) - 1)
    def _fin():
        st_ref[...] = acc_ref[...]


def _mlp1(x, h1, h2, h3, w1, bb1):
    return pl.pallas_call(
        _mlp1_body,
        grid=(NN // RB,),
        in_specs=[
            pl.BlockSpec((RB, DD), lambda i: (i, 0)),
            pl.BlockSpec((RB, DD), lambda i: (i, 0)),
            pl.BlockSpec((RB, DD), lambda i: (i, 0)),
            pl.BlockSpec((RB, DD), lambda i: (i, 0)),
            pl.BlockSpec((4 * DD, 2 * CC), lambda i: (0, 0)),
            pl.BlockSpec((1, 2 * CC), lambda i: (0, 0)),
        ],
        out_specs=[
            pl.BlockSpec((RB, 2 * CC), lambda i: (i, 0)),
            pl.BlockSpec((2, 2 * CC), lambda i: (0, 0)),
        ],
        out_shape=[
            jax.ShapeDtypeStruct((NN, 2 * CC), jnp.float32),
            jax.ShapeDtypeStruct((2, 2 * CC), jnp.float32),
        ],
        scratch_shapes=[pltpu.VMEM((2, 2 * CC), jnp.float32)],
    )(x, h1, h2, h3, w1, bb1.reshape(1, 2 * CC))


def _mlp2_body(z_ref, st_ref, gm_ref, bm_ref, w2_ref, bb2_ref, o_ref):
    mu = st_ref[0:1, :] / NN
    var = st_ref[1:2, :] / NN - mu * mu
    scale = gm_ref[...] * lax.rsqrt(var + 1e-5)
    t = (z_ref[...] - mu) * scale + bm_ref[...]
    t = jnp.where(t >= 0.0, t, 0.01 * t)
    o_ref[...] = jnp.dot(t, w2_ref[...], preferred_element_type=jnp.float32) + bb2_ref[...]


def _mlp2(z, st, gm, bm, w2, bb2):
    return pl.pallas_call(
        _mlp2_body,
        grid=(NN // RB,),
        in_specs=[
            pl.BlockSpec((RB, 2 * CC), lambda i: (i, 0)),
            pl.BlockSpec((2, 2 * CC), lambda i: (0, 0)),
            pl.BlockSpec((1, 2 * CC), lambda i: (0, 0)),
            pl.BlockSpec((1, 2 * CC), lambda i: (0, 0)),
            pl.BlockSpec((2 * CC, CC), lambda i: (0, 0)),
            pl.BlockSpec((1, CC), lambda i: (0, 0)),
        ],
        out_specs=pl.BlockSpec((RB, CC), lambda i: (i, 0)),
        out_shape=jax.ShapeDtypeStruct((NN, CC), jnp.float32),
    )(z, st, gm.reshape(1, 2 * CC), bm.reshape(1, 2 * CC), w2, bb2.reshape(1, CC))


def kernel(x, edge_index, Wl0, Wr0, Wl1, Wr1, Wl2, Wr2,
           g0, b0, g1, b1, g2, b2, W1, bb1, gm, bm, W2, bb2):
    src = edge_index[0]
    dst = edge_index[1]
    pad = EPAD - EE
    pad_src = jnp.arange(pad, dtype=jnp.int32) % NN
    src2d = jnp.concatenate([src, pad_src]).reshape(EPAD // CHUNK, CHUNK)
    pad_dst = NN + (jnp.arange(pad, dtype=jnp.int32) % (NACC - NN))
    dst2d = jnp.concatenate([dst, pad_dst]).reshape(EPAD // CHUNK, CHUNK)
    zstripe = jnp.zeros((STRIPE, DD), jnp.float32)
    ones = jnp.ones((CHUNK, DD), jnp.float32)

    sc_cnt = _get_sc_cnt()
    sc_agg = _get_sc_agg()
    pc = sc_cnt(dst2d, zstripe, ones)
    pf0 = sc_agg(x, src2d, dst2d, zstripe)
    y1, st1 = _conv(x, pf0, pc, Wl0, Wr0)
    h1 = _bnrelu(y1, st1, g0, b0)

    pf1 = sc_agg(h1, src2d, dst2d, zstripe)
    y2, st2 = _conv(h1, pf1, pc, Wl1, Wr1)
    h2 = _bnrelu(y2, st2, g1, b1)

    pf2 = sc_agg(h2, src2d, dst2d, zstripe)
    y3, st3 = _conv(h2, pf2, pc, Wl2, Wr2)
    h3 = _bnrelu(y3, st3, g2, b2)

    z, stz = _mlp1(x, h1, h2, h3, W1, bb1)
    return _mlp2(z, stz, gm, bm, W2, bb2)


# speedup vs baseline: 7.9215x; 1.1953x over previous
import functools

import jax
import jax.numpy as jnp
from jax import lax
from jax.experimental import pallas as pl
from jax.experimental.pallas import tpu as pltpu
from jax.experimental.pallas import tpu_sc as plsc

NN = 10000
EE = 320000
DD = 128
CC = 64

NC, NS = 2, 16
NW = NC * NS
CHUNK = 128
KCH = 80
EPT = KCH * CHUNK
EPAD = EPT * NW
NACC = 10240
STRIPE = NACC // NS
DUMMY = 10016
CW = 16
GB = 16


@functools.lru_cache(maxsize=None)
def _get_sc_agg():
  mesh = plsc.VectorSubcoreMesh(core_axis_name="c", subcore_axis_name="s")

  @functools.partial(
      pl.kernel, mesh=mesh,
      out_type=jax.ShapeDtypeStruct((NC, NACC, DD), jnp.float32),
      scratch_types=[
          pltpu.VMEM((GB, CHUNK), jnp.int32),
          pltpu.VMEM((GB, CHUNK), jnp.int32),
          pltpu.VMEM((CHUNK, DD), jnp.float32),
          pltpu.VMEM((CHUNK, DD), jnp.float32),
          pltpu.VMEM_SHARED((NACC, DD), jnp.float32),
          pltpu.SemaphoreType.DMA,
          pltpu.SemaphoreType.DMA,
      ],
  )
  def _sc_agg(h_hbm, src_hbm, dst_hbm, zstripe_hbm,
              out_f, src_v, dst_v, rows_a, rows_b, accf, sem_a, sem_b):
    c = lax.axis_index("c")
    s = lax.axis_index("s")
    wid = c * NS + s

    pltpu.sync_copy(zstripe_hbm, accf.at[pl.ds(s * STRIPE, STRIPE)])
    plsc.subcore_barrier()

    def window(b, carry):
        base = wid * KCH + b * GB
        pltpu.sync_copy(src_hbm.at[pl.ds(base, GB)], src_v)
        pltpu.sync_copy(dst_hbm.at[pl.ds(base, GB)], dst_v)
        pltpu.async_copy(h_hbm.at[src_v.at[0]], rows_a, sem_a)

        def step(t, carry2):
            pltpu.make_async_copy(h_hbm.at[src_v.at[2 * t]], rows_a,
                                  sem_a).wait()
            pltpu.async_copy(h_hbm.at[src_v.at[2 * t + 1]], rows_b, sem_b)
            pltpu.sync_copy(rows_a, accf.at[dst_v.at[2 * t]], add=True)
            pltpu.make_async_copy(h_hbm.at[src_v.at[2 * t + 1]], rows_b,
                                  sem_b).wait()

            @pl.when(t < GB // 2 - 1)
            def _nxt():
                pltpu.async_copy(h_hbm.at[src_v.at[2 * t + 2]], rows_a, sem_a)
            pltpu.sync_copy(rows_b, accf.at[dst_v.at[2 * t + 1]], add=True)
            return carry2
        lax.fori_loop(0, GB // 2, step, 0)
        return carry
    lax.fori_loop(0, KCH // GB, window, 0)

    plsc.subcore_barrier()
    pltpu.sync_copy(accf.at[pl.ds(s * STRIPE, STRIPE)],
                    out_f.at[c, pl.ds(s * STRIPE, STRIPE)])

  return _sc_agg


@functools.lru_cache(maxsize=None)
def _get_sc_cnt():
  mesh = plsc.VectorSubcoreMesh(core_axis_name="c", subcore_axis_name="s")

  @functools.partial(
      pl.kernel, mesh=mesh,
      out_type=jax.ShapeDtypeStruct((NC, NACC, DD), jnp.float32),
      scratch_types=[
          pltpu.VMEM((KCH, CHUNK), jnp.int32),
          pltpu.VMEM((CHUNK, DD), jnp.float32),
          pltpu.VMEM_SHARED((NACC, DD), jnp.float32),
      ],
  )
  def _sc_cnt(dst_hbm, zstripe_hbm, ones_hbm, out_c, dst_v, ones_v, accc):
    c = lax.axis_index("c")
    s = lax.axis_index("s")
    wid = c * NS + s

    pltpu.sync_copy(zstripe_hbm, accc.at[pl.ds(s * STRIPE, STRIPE)])
    pltpu.sync_copy(ones_hbm, ones_v)
    pltpu.sync_copy(dst_hbm.at[pl.ds(wid * KCH, KCH)], dst_v)
    plsc.subcore_barrier()

    def step(j, carry):
        pltpu.sync_copy(ones_v, accc.at[dst_v.at[j]], add=True)
        return carry
    lax.fori_loop(0, KCH, step, 0)

    plsc.subcore_barrier()
    pltpu.sync_copy(accc.at[pl.ds(s * STRIPE, STRIPE)],
                    out_c.at[c, pl.ds(s * STRIPE, STRIPE)])

  return _sc_cnt


RB = 1000


def _conv_body(h_ref, pf_ref, pc_ref, wl_ref, wr_ref, y_ref, st_ref, acc_ref):
    i = pl.program_id(0)

    @pl.when(i == 0)
    def _init():
        acc_ref[...] = jnp.zeros_like(acc_ref)

    psum = pf_ref[0] + pf_ref[1]
    cnt = pc_ref[0, :, 0:1] + pc_ref[1, :, 0:1]
    inv = 1.0 / jnp.maximum(cnt, 1.0)
    m = jnp.dot(psum, wl_ref[...], preferred_element_type=jnp.float32) * inv
    y = m + jnp.dot(h_ref[...], wr_ref[...], preferred_element_type=jnp.float32)
    y_ref[...] = y
    acc_ref[0:1, :] += jnp.sum(y, axis=0, keepdims=True)
    acc_ref[1:2, :] += jnp.sum(y * y, axis=0, keepdims=True)

    @pl.when(i == pl.num_programs(0) - 1)
    def _fin():
        st_ref[...] = acc_ref[...]


def _conv(h, pf, pc, wl, wr):
    return pl.pallas_call(
        _conv_body,
        grid=(NN // RB,),
        in_specs=[
            pl.BlockSpec((RB, DD), lambda i: (i, 0)),
            pl.BlockSpec((NC, RB, DD), lambda i: (0, i, 0)),
            pl.BlockSpec((NC, RB, DD), lambda i: (0, i, 0)),
            pl.BlockSpec((DD, DD), lambda i: (0, 0)),
            pl.BlockSpec((DD, DD), lambda i: (0, 0)),
        ],
        out_specs=[
            pl.BlockSpec((RB, DD), lambda i: (i, 0)),
            pl.BlockSpec((2, DD), lambda i: (0, 0)),
        ],
        out_shape=[
            jax.ShapeDtypeStruct((NN, DD), jnp.float32),
            jax.ShapeDtypeStruct((2, DD), jnp.float32),
        ],
        scratch_shapes=[pltpu.VMEM((2, DD), jnp.float32)],
    )(h, pf, pc, wl, wr)


def _bnrelu_body(y_ref, st_ref, g_ref, b_ref, o_ref):
    mu = st_ref[0:1, :] / NN
    var = st_ref[1:2, :] / NN - mu * mu
    scale = g_ref[...] * lax.rsqrt(var + 1e-5)
    o_ref[...] = jnp.maximum((y_ref[...] - mu) * scale + b_ref[...], 0.0)


def _bnrelu(y, st, g, b):
    return pl.pallas_call(
        _bnrelu_body,
        grid=(NN // RB,),
        in_specs=[
            pl.BlockSpec((RB, DD), lambda i: (i, 0)),
            pl.BlockSpec((2, DD), lambda i: (0, 0)),
            pl.BlockSpec((1, DD), lambda i: (0, 0)),
            pl.BlockSpec((1, DD), lambda i: (0, 0)),
        ],
        out_specs=pl.BlockSpec((RB, DD), lambda i: (i, 0)),
        out_shape=jax.ShapeDtypeStruct((NN, DD), jnp.float32),
    )(y, st, g.reshape(1, DD), b.reshape(1, DD))


def _mlp1_body(x_ref, h1_ref, h2_ref, h3_ref, w1_ref, bb1_ref,
               z_ref, st_ref, acc_ref):
    i = pl.program_id(0)

    @pl.when(i == 0)
    def _init():
        acc_ref[...] = jnp.zeros_like(acc_ref)

    z = (jnp.dot(x_ref[...], w1_ref[0 * DD:1 * DD], preferred_element_type=jnp.float32)
         + jnp.dot(h1_ref[...], w1_ref[1 * DD:2 * DD], preferred_element_type=jnp.float32)
         + jnp.dot(h2_ref[...], w1_ref[2 * DD:3 * DD], preferred_element_type=jnp.float32)
         + jnp.dot(h3_ref[...], w1_ref[3 * DD:4 * DD], preferred_element_type=jnp.float32)
         + bb1_ref[...])
    z_ref[...] = z
    acc_ref[0:1, :] += jnp.sum(z, axis=0, keepdims=True)
    acc_ref[1:2, :] += jnp.sum(z * z, axis=0, keepdims=True)

    @pl.when(i == pl.num_programs(0) - 1)
    def _fin():
        st_ref[...] = acc_ref[...]


def _mlp1(x, h1, h2, h3, w1, bb1):
    return pl.pallas_call(
        _mlp1_body,
        grid=(NN // RB,),
        in_specs=[
            pl.BlockSpec((RB, DD), lambda i: (i, 0)),
            pl.BlockSpec((RB, DD), lambda i: (i, 0)),
            pl.BlockSpec((RB, DD), lambda i: (i, 0)),
            pl.BlockSpec((RB, DD), lambda i: (i, 0)),
            pl.BlockSpec((4 * DD, 2 * CC), lambda i: (0, 0)),
            pl.BlockSpec((1, 2 * CC), lambda i: (0, 0)),
        ],
        out_specs=[
            pl.BlockSpec((RB, 2 * CC), lambda i: (i, 0)),
            pl.BlockSpec((2, 2 * CC), lambda i: (0, 0)),
        ],
        out_shape=[
            jax.ShapeDtypeStruct((NN, 2 * CC), jnp.float32),
            jax.ShapeDtypeStruct((2, 2 * CC), jnp.float32),
        ],
        scratch_shapes=[pltpu.VMEM((2, 2 * CC), jnp.float32)],
    )(x, h1, h2, h3, w1, bb1.reshape(1, 2 * CC))


def _mlp2_body(z_ref, st_ref, gm_ref, bm_ref, w2_ref, bb2_ref, o_ref):
    mu = st_ref[0:1, :] / NN
    var = st_ref[1:2, :] / NN - mu * mu
    scale = gm_ref[...] * lax.rsqrt(var + 1e-5)
    t = (z_ref[...] - mu) * scale + bm_ref[...]
    t = jnp.where(t >= 0.0, t, 0.01 * t)
    o_ref[...] = jnp.dot(t, w2_ref[...], preferred_element_type=jnp.float32) + bb2_ref[...]


def _mlp2(z, st, gm, bm, w2, bb2):
    return pl.pallas_call(
        _mlp2_body,
        grid=(NN // RB,),
        in_specs=[
            pl.BlockSpec((RB, 2 * CC), lambda i: (i, 0)),
            pl.BlockSpec((2, 2 * CC), lambda i: (0, 0)),
            pl.BlockSpec((1, 2 * CC), lambda i: (0, 0)),
            pl.BlockSpec((1, 2 * CC), lambda i: (0, 0)),
            pl.BlockSpec((2 * CC, CC), lambda i: (0, 0)),
            pl.BlockSpec((1, CC), lambda i: (0, 0)),
        ],
        out_specs=pl.BlockSpec((RB, CC), lambda i: (i, 0)),
        out_shape=jax.ShapeDtypeStruct((NN, CC), jnp.float32),
    )(z, st, gm.reshape(1, 2 * CC), bm.reshape(1, 2 * CC), w2, bb2.reshape(1, CC))


def kernel(x, edge_index, Wl0, Wr0, Wl1, Wr1, Wl2, Wr2,
           g0, b0, g1, b1, g2, b2, W1, bb1, gm, bm, W2, bb2):
    src = edge_index[0]
    dst = edge_index[1]
    pad = EPAD - EE
    pad_src = jnp.arange(pad, dtype=jnp.int32) % NN
    src2d = jnp.concatenate([src, pad_src]).reshape(EPAD // CHUNK, CHUNK)
    pad_dst = NN + (jnp.arange(pad, dtype=jnp.int32) % (NACC - NN))
    dst2d = jnp.concatenate([dst, pad_dst]).reshape(EPAD // CHUNK, CHUNK)
    zstripe = jnp.zeros((STRIPE, DD), jnp.float32)
    ones = jnp.ones((CHUNK, DD), jnp.float32)

    sc_cnt = _get_sc_cnt()
    sc_agg = _get_sc_agg()
    pc = sc_cnt(dst2d, zstripe, ones)
    pf0 = sc_agg(x, src2d, dst2d, zstripe)
    y1, st1 = _conv(x, pf0, pc, Wl0, Wr0)
    h1 = _bnrelu(y1, st1, g0, b0)

    pf1 = sc_agg(h1, src2d, dst2d, zstripe)
    y2, st2 = _conv(h1, pf1, pc, Wl1, Wr1)
    h2 = _bnrelu(y2, st2, g1, b1)

    pf2 = sc_agg(h2, src2d, dst2d, zstripe)
    y3, st3 = _conv(h2, pf2, pc, Wl2, Wr2)
    h3 = _bnrelu(y3, st3, g2, b2)

    z, stz = _mlp1(x, h1, h2, h3, W1, bb1)
    return _mlp2(z, stz, gm, bm, W2, bb2)


# speedup vs baseline: 8.0162x; 1.0120x over previous
import functools

import jax
import jax.numpy as jnp
from jax import lax
from jax.experimental import pallas as pl
from jax.experimental.pallas import tpu as pltpu
from jax.experimental.pallas import tpu_sc as plsc

NN = 10000
EE = 320000
DD = 128
CC = 64

NC, NS = 2, 16
NW = NC * NS
CHUNK = 128
KCH = 80
EPT = KCH * CHUNK
EPAD = EPT * NW
NACC = 10240
STRIPE = NACC // NS
DUMMY = 10016
CW = 16
GB = 16


@functools.lru_cache(maxsize=None)
def _get_sc_agg():
  mesh = plsc.VectorSubcoreMesh(core_axis_name="c", subcore_axis_name="s")

  @functools.partial(
      pl.kernel, mesh=mesh,
      out_type=jax.ShapeDtypeStruct((NC, NACC, DD), jnp.float32),
      scratch_types=[
          pltpu.VMEM((GB, CHUNK), jnp.int32),
          pltpu.VMEM((GB, CHUNK), jnp.int32),
          pltpu.VMEM((CHUNK, DD), jnp.float32),
          pltpu.VMEM((CHUNK, DD), jnp.float32),
          pltpu.VMEM_SHARED((NACC, DD), jnp.float32),
          pltpu.SemaphoreType.DMA,
          pltpu.SemaphoreType.DMA,
      ],
  )
  def _sc_agg(h_hbm, src_hbm, dst_hbm, zstripe_hbm,
              out_f, src_v, dst_v, rows_a, rows_b, accf, sem_a, sem_b):
    c = lax.axis_index("c")
    s = lax.axis_index("s")
    wid = c * NS + s

    pltpu.sync_copy(zstripe_hbm, accf.at[pl.ds(s * STRIPE, STRIPE)])
    plsc.subcore_barrier()

    def window(b, carry):
        base = wid * KCH + b * GB
        pltpu.sync_copy(src_hbm.at[pl.ds(base, GB)], src_v)
        pltpu.sync_copy(dst_hbm.at[pl.ds(base, GB)], dst_v)
        pltpu.async_copy(h_hbm.at[src_v.at[0]], rows_a, sem_a)

        def step(t, carry2):
            pltpu.make_async_copy(h_hbm.at[src_v.at[2 * t]], rows_a,
                                  sem_a).wait()
            pltpu.async_copy(h_hbm.at[src_v.at[2 * t + 1]], rows_b, sem_b)
            pltpu.sync_copy(rows_a, accf.at[dst_v.at[2 * t]], add=True)
            pltpu.make_async_copy(h_hbm.at[src_v.at[2 * t + 1]], rows_b,
                                  sem_b).wait()

            @pl.when(t < GB // 2 - 1)
            def _nxt():
                pltpu.async_copy(h_hbm.at[src_v.at[2 * t + 2]], rows_a, sem_a)
            pltpu.sync_copy(rows_b, accf.at[dst_v.at[2 * t + 1]], add=True)
            return carry2
        lax.fori_loop(0, GB // 2, step, 0)
        return carry
    lax.fori_loop(0, KCH // GB, window, 0)

    plsc.subcore_barrier()
    pltpu.sync_copy(accf.at[pl.ds(s * STRIPE, STRIPE)],
                    out_f.at[c, pl.ds(s * STRIPE, STRIPE)])

  return _sc_agg


@functools.lru_cache(maxsize=None)
def _get_sc_cnt():
  mesh = plsc.VectorSubcoreMesh(core_axis_name="c", subcore_axis_name="s")

  @functools.partial(
      pl.kernel, mesh=mesh,
      out_type=jax.ShapeDtypeStruct((NC, NACC, DD), jnp.float32),
      scratch_types=[
          pltpu.VMEM((KCH, CHUNK), jnp.int32),
          pltpu.VMEM((CHUNK, DD), jnp.float32),
          pltpu.VMEM_SHARED((NACC, DD), jnp.float32),
      ],
  )
  def _sc_cnt(dst_hbm, zstripe_hbm, ones_hbm, out_c, dst_v, ones_v, accc):
    c = lax.axis_index("c")
    s = lax.axis_index("s")
    wid = c * NS + s

    pltpu.sync_copy(zstripe_hbm, accc.at[pl.ds(s * STRIPE, STRIPE)])
    pltpu.sync_copy(ones_hbm, ones_v)
    pltpu.sync_copy(dst_hbm.at[pl.ds(wid * KCH, KCH)], dst_v)
    plsc.subcore_barrier()

    def step(j, carry):
        pltpu.sync_copy(ones_v, accc.at[dst_v.at[j]], add=True)
        return carry
    lax.fori_loop(0, KCH, step, 0)

    plsc.subcore_barrier()
    pltpu.sync_copy(accc.at[pl.ds(s * STRIPE, STRIPE)],
                    out_c.at[c, pl.ds(s * STRIPE, STRIPE)])

  return _sc_cnt


RB = 1000


def _conv_body(h_ref, pf_ref, pc_ref, wl_ref, wr_ref, y_ref, st_ref, acc_ref):
    i = pl.program_id(0)

    @pl.when(i == 0)
    def _init():
        acc_ref[...] = jnp.zeros_like(acc_ref)

    psum = pf_ref[0] + pf_ref[1]
    cnt = pc_ref[0, :, 0:1] + pc_ref[1, :, 0:1]
    inv = 1.0 / jnp.maximum(cnt, 1.0)
    m = jnp.dot(psum, wl_ref[...], preferred_element_type=jnp.float32) * inv
    y = m + jnp.dot(h_ref[...], wr_ref[...], preferred_element_type=jnp.float32)
    y_ref[...] = y
    acc_ref[0:1, :] += jnp.sum(y, axis=0, keepdims=True)
    acc_ref[1:2, :] += jnp.sum(y * y, axis=0, keepdims=True)

    @pl.when(i == pl.num_programs(0) - 1)
    def _fin():
        st_ref[...] = acc_ref[...]


def _conv(h, pf, pc, wl, wr):
    return pl.pallas_call(
        _conv_body,
        grid=(NN // RB,),
        in_specs=[
            pl.BlockSpec((RB, DD), lambda i: (i, 0)),
            pl.BlockSpec((NC, RB, DD), lambda i: (0, i, 0)),
            pl.BlockSpec((NC, RB, DD), lambda i: (0, i, 0)),
            pl.BlockSpec((DD, DD), lambda i: (0, 0)),
            pl.BlockSpec((DD, DD), lambda i: (0, 0)),
        ],
        out_specs=[
            pl.BlockSpec((RB, DD), lambda i: (i, 0)),
            pl.BlockSpec((2, DD), lambda i: (0, 0)),
        ],
        out_shape=[
            jax.ShapeDtypeStruct((NN, DD), jnp.float32),
            jax.ShapeDtypeStruct((2, DD), jnp.float32),
        ],
        scratch_shapes=[pltpu.VMEM((2, DD), jnp.float32)],
    )(h, pf, pc, wl, wr)


def _bnrelu_body(y_ref, st_ref, g_ref, b_ref, o_ref):
    mu = st_ref[0:1, :] / NN
    var = st_ref[1:2, :] / NN - mu * mu
    scale = g_ref[...] * lax.rsqrt(var + 1e-5)
    o_ref[...] = jnp.maximum((y_ref[...] - mu) * scale + b_ref[...], 0.0)


def _bnrelu(y, st, g, b):
    return pl.pallas_call(
        _bnrelu_body,
        grid=(NN // RB,),
        in_specs=[
            pl.BlockSpec((RB, DD), lambda i: (i, 0)),
            pl.BlockSpec((2, DD), lambda i: (0, 0)),
            pl.BlockSpec((1, DD), lambda i: (0, 0)),
            pl.BlockSpec((1, DD), lambda i: (0, 0)),
        ],
        out_specs=pl.BlockSpec((RB, DD), lambda i: (i, 0)),
        out_shape=jax.ShapeDtypeStruct((NN, DD), jnp.float32),
    )(y, st, g.reshape(1, DD), b.reshape(1, DD))


def _mlp1_body(x_ref, h1_ref, h2_ref, y3_ref, st3_ref, g2_ref, b2_ref,
               w1_ref, bb1_ref, z_ref, st_ref, acc_ref):
    i = pl.program_id(0)

    @pl.when(i == 0)
    def _init():
        acc_ref[...] = jnp.zeros_like(acc_ref)

    mu3 = st3_ref[0:1, :] / NN
    var3 = st3_ref[1:2, :] / NN - mu3 * mu3
    sc3 = g2_ref[...] * lax.rsqrt(var3 + 1e-5)
    h3 = jnp.maximum((y3_ref[...] - mu3) * sc3 + b2_ref[...], 0.0)
    z = (jnp.dot(x_ref[...], w1_ref[0 * DD:1 * DD], preferred_element_type=jnp.float32)
         + jnp.dot(h1_ref[...], w1_ref[1 * DD:2 * DD], preferred_element_type=jnp.float32)
         + jnp.dot(h2_ref[...], w1_ref[2 * DD:3 * DD], preferred_element_type=jnp.float32)
         + jnp.dot(h3, w1_ref[3 * DD:4 * DD], preferred_element_type=jnp.float32)
         + bb1_ref[...])
    z_ref[...] = z
    acc_ref[0:1, :] += jnp.sum(z, axis=0, keepdims=True)
    acc_ref[1:2, :] += jnp.sum(z * z, axis=0, keepdims=True)

    @pl.when(i == pl.num_programs(0) - 1)
    def _fin():
        st_ref[...] = acc_ref[...]


def _mlp1(x, h1, h2, y3, st3, g2, b2, w1, bb1):
    return pl.pallas_call(
        _mlp1_body,
        grid=(NN // RB,),
        in_specs=[
            pl.BlockSpec((RB, DD), lambda i: (i, 0)),
            pl.BlockSpec((RB, DD), lambda i: (i, 0)),
            pl.BlockSpec((RB, DD), lambda i: (i, 0)),
            pl.BlockSpec((RB, DD), lambda i: (i, 0)),
            pl.BlockSpec((2, DD), lambda i: (0, 0)),
            pl.BlockSpec((1, DD), lambda i: (0, 0)),
            pl.BlockSpec((1, DD), lambda i: (0, 0)),
            pl.BlockSpec((4 * DD, 2 * CC), lambda i: (0, 0)),
            pl.BlockSpec((1, 2 * CC), lambda i: (0, 0)),
        ],
        out_specs=[
            pl.BlockSpec((RB, 2 * CC), lambda i: (i, 0)),
            pl.BlockSpec((2, 2 * CC), lambda i: (0, 0)),
        ],
        out_shape=[
            jax.ShapeDtypeStruct((NN, 2 * CC), jnp.float32),
            jax.ShapeDtypeStruct((2, 2 * CC), jnp.float32),
        ],
        scratch_shapes=[pltpu.VMEM((2, 2 * CC), jnp.float32)],
    )(x, h1, h2, y3, st3, g2.reshape(1, DD), b2.reshape(1, DD), w1,
      bb1.reshape(1, 2 * CC))


def _mlp2_body(z_ref, st_ref, gm_ref, bm_ref, w2_ref, bb2_ref, o_ref):
    mu = st_ref[0:1, :] / NN
    var = st_ref[1:2, :] / NN - mu * mu
    scale = gm_ref[...] * lax.rsqrt(var + 1e-5)
    t = (z_ref[...] - mu) * scale + bm_ref[...]
    t = jnp.where(t >= 0.0, t, 0.01 * t)
    o_ref[...] = jnp.dot(t, w2_ref[...], preferred_element_type=jnp.float32) + bb2_ref[...]


def _mlp2(z, st, gm, bm, w2, bb2):
    return pl.pallas_call(
        _mlp2_body,
        grid=(NN // RB,),
        in_specs=[
            pl.BlockSpec((RB, 2 * CC), lambda i: (i, 0)),
            pl.BlockSpec((2, 2 * CC), lambda i: (0, 0)),
            pl.BlockSpec((1, 2 * CC), lambda i: (0, 0)),
            pl.BlockSpec((1, 2 * CC), lambda i: (0, 0)),
            pl.BlockSpec((2 * CC, CC), lambda i: (0, 0)),
            pl.BlockSpec((1, CC), lambda i: (0, 0)),
        ],
        out_specs=pl.BlockSpec((RB, CC), lambda i: (i, 0)),
        out_shape=jax.ShapeDtypeStruct((NN, CC), jnp.float32),
    )(z, st, gm.reshape(1, 2 * CC), bm.reshape(1, 2 * CC), w2, bb2.reshape(1, CC))


def kernel(x, edge_index, Wl0, Wr0, Wl1, Wr1, Wl2, Wr2,
           g0, b0, g1, b1, g2, b2, W1, bb1, gm, bm, W2, bb2):
    src = edge_index[0]
    dst = edge_index[1]
    pad = EPAD - EE
    pad_src = jnp.arange(pad, dtype=jnp.int32) % NN
    src2d = jnp.concatenate([src, pad_src]).reshape(EPAD // CHUNK, CHUNK)
    pad_dst = NN + (jnp.arange(pad, dtype=jnp.int32) % (NACC - NN))
    dst2d = jnp.concatenate([dst, pad_dst]).reshape(EPAD // CHUNK, CHUNK)
    zstripe = jnp.zeros((STRIPE, DD), jnp.float32)
    ones = jnp.ones((CHUNK, DD), jnp.float32)

    sc_cnt = _get_sc_cnt()
    sc_agg = _get_sc_agg()
    pc = sc_cnt(dst2d, zstripe, ones)
    pf0 = sc_agg(x, src2d, dst2d, zstripe)
    y1, st1 = _conv(x, pf0, pc, Wl0, Wr0)
    h1 = _bnrelu(y1, st1, g0, b0)

    pf1 = sc_agg(h1, src2d, dst2d, zstripe)
    y2, st2 = _conv(h1, pf1, pc, Wl1, Wr1)
    h2 = _bnrelu(y2, st2, g1, b1)

    pf2 = sc_agg(h2, src2d, dst2d, zstripe)
    y3, st3 = _conv(h2, pf2, pc, Wl2, Wr2)

    z, stz = _mlp1(x, h1, h2, y3, st3, g2, b2, W1, bb1)
    return _mlp2(z, stz, gm, bm, W2, bb2)


# speedup vs baseline: 8.0283x; 1.0015x over previous
import functools

import jax
import jax.numpy as jnp
from jax import lax
from jax.experimental import pallas as pl
from jax.experimental.pallas import tpu as pltpu
from jax.experimental.pallas import tpu_sc as plsc

NN = 10000
EE = 320000
DD = 128
CC = 64

NC, NS = 2, 16
NW = NC * NS
CHUNK = 128
KCH = 80
EPT = KCH * CHUNK
EPAD = EPT * NW
NACC = 10240
STRIPE = NACC // NS
GB = 16


@functools.lru_cache(maxsize=None)
def _get_sc_agg():
  mesh = plsc.VectorSubcoreMesh(core_axis_name="c", subcore_axis_name="s")

  @functools.partial(
      pl.kernel, mesh=mesh,
      out_type=jax.ShapeDtypeStruct((NC, NACC, DD), jnp.float32),
      scratch_types=[
          pltpu.VMEM((GB, CHUNK), jnp.int32),
          pltpu.VMEM((GB, CHUNK), jnp.int32),
          pltpu.VMEM((CHUNK, DD), jnp.float32),
          pltpu.VMEM((CHUNK, DD), jnp.float32),
          pltpu.VMEM_SHARED((NACC, DD), jnp.float32),
          pltpu.SemaphoreType.DMA,
          pltpu.SemaphoreType.DMA,
      ],
  )
  def _sc_agg(h_hbm, src_hbm, dst_hbm, zstripe_hbm,
              out_f, src_v, dst_v, rows_a, rows_b, accf, sem_a, sem_b):
    c = lax.axis_index("c")
    s = lax.axis_index("s")
    wid = c * NS + s

    pltpu.sync_copy(zstripe_hbm, accf.at[pl.ds(s * STRIPE, STRIPE)])
    plsc.subcore_barrier()

    def window(b, carry):
        base = wid * KCH + b * GB
        pltpu.sync_copy(src_hbm.at[pl.ds(base, GB)], src_v)
        pltpu.sync_copy(dst_hbm.at[pl.ds(base, GB)], dst_v)
        pltpu.async_copy(h_hbm.at[src_v.at[0]], rows_a, sem_a)

        def step(t, carry2):
            pltpu.make_async_copy(h_hbm.at[src_v.at[2 * t]], rows_a,
                                  sem_a).wait()
            pltpu.async_copy(h_hbm.at[src_v.at[2 * t + 1]], rows_b, sem_b)
            pltpu.sync_copy(rows_a, accf.at[dst_v.at[2 * t]], add=True)
            pltpu.make_async_copy(h_hbm.at[src_v.at[2 * t + 1]], rows_b,
                                  sem_b).wait()

            @pl.when(t < GB // 2 - 1)
            def _nxt():
                pltpu.async_copy(h_hbm.at[src_v.at[2 * t + 2]], rows_a, sem_a)
            pltpu.sync_copy(rows_b, accf.at[dst_v.at[2 * t + 1]], add=True)
            return carry2
        lax.fori_loop(0, GB // 2, step, 0)
        return carry
    lax.fori_loop(0, KCH // GB, window, 0)

    plsc.subcore_barrier()
    pltpu.sync_copy(accf.at[pl.ds(s * STRIPE, STRIPE)],
                    out_f.at[c, pl.ds(s * STRIPE, STRIPE)])

  return _sc_agg


@functools.lru_cache(maxsize=None)
def _get_sc_cnt():
  mesh = plsc.VectorSubcoreMesh(core_axis_name="c", subcore_axis_name="s")

  @functools.partial(
      pl.kernel, mesh=mesh,
      out_type=jax.ShapeDtypeStruct((NC, NACC, DD), jnp.float32),
      scratch_types=[
          pltpu.VMEM((KCH, CHUNK), jnp.int32),
          pltpu.VMEM((CHUNK, DD), jnp.float32),
          pltpu.VMEM_SHARED((NACC, DD), jnp.float32),
      ],
  )
  def _sc_cnt(dst_hbm, zstripe_hbm, ones_hbm, out_c, dst_v, ones_v, accc):
    c = lax.axis_index("c")
    s = lax.axis_index("s")
    wid = c * NS + s

    pltpu.sync_copy(zstripe_hbm, accc.at[pl.ds(s * STRIPE, STRIPE)])
    pltpu.sync_copy(ones_hbm, ones_v)
    pltpu.sync_copy(dst_hbm.at[pl.ds(wid * KCH, KCH)], dst_v)
    plsc.subcore_barrier()

    def step(j, carry):
        pltpu.sync_copy(ones_v, accc.at[dst_v.at[j]], add=True)
        return carry
    lax.fori_loop(0, KCH, step, 0)

    plsc.subcore_barrier()
    pltpu.sync_copy(accc.at[pl.ds(s * STRIPE, STRIPE)],
                    out_c.at[c, pl.ds(s * STRIPE, STRIPE)])

  return _sc_cnt


RB = 1000


def _conv_body(h_ref, pf_ref, pc_ref, wl_ref, wr_ref, y_ref, st_ref, acc_ref):
    i = pl.program_id(0)

    @pl.when(i == 0)
    def _init():
        acc_ref[...] = jnp.zeros_like(acc_ref)

    psum = pf_ref[0] + pf_ref[1]
    cnt = pc_ref[0, :, 0:1] + pc_ref[1, :, 0:1]
    inv = 1.0 / jnp.maximum(cnt, 1.0)
    m = jnp.dot(psum, wl_ref[...], preferred_element_type=jnp.float32) * inv
    y = m + jnp.dot(h_ref[...], wr_ref[...], preferred_element_type=jnp.float32)
    y_ref[...] = y
    acc_ref[0:1, :] += jnp.sum(y, axis=0, keepdims=True)
    acc_ref[1:2, :] += jnp.sum(y * y, axis=0, keepdims=True)

    @pl.when(i == pl.num_programs(0) - 1)
    def _fin():
        st_ref[...] = acc_ref[...]


def _conv(h, pf, pc, wl, wr):
    return pl.pallas_call(
        _conv_body,
        grid=(NN // RB,),
        in_specs=[
            pl.BlockSpec((RB, DD), lambda i: (i, 0)),
            pl.BlockSpec((NC, RB, DD), lambda i: (0, i, 0)),
            pl.BlockSpec((NC, RB, DD), lambda i: (0, i, 0)),
            pl.BlockSpec((DD, DD), lambda i: (0, 0)),
            pl.BlockSpec((DD, DD), lambda i: (0, 0)),
        ],
        out_specs=[
            pl.BlockSpec((RB, DD), lambda i: (i, 0)),
            pl.BlockSpec((2, DD), lambda i: (0, 0)),
        ],
        out_shape=[
            jax.ShapeDtypeStruct((NN, DD), jnp.float32),
            jax.ShapeDtypeStruct((2, DD), jnp.float32),
        ],
        scratch_shapes=[pltpu.VMEM((2, DD), jnp.float32)],
    )(h, pf, pc, wl, wr)


def _bnrelu_body(y_ref, st_ref, g_ref, b_ref, o_ref):
    mu = st_ref[0:1, :] / NN
    var = st_ref[1:2, :] / NN - mu * mu
    scale = g_ref[...] * lax.rsqrt(var + 1e-5)
    o_ref[...] = jnp.maximum((y_ref[...] - mu) * scale + b_ref[...], 0.0)


def _bnrelu(y, st, g, b):
    return pl.pallas_call(
        _bnrelu_body,
        grid=(NN // RB,),
        in_specs=[
            pl.BlockSpec((RB, DD), lambda i: (i, 0)),
            pl.BlockSpec((2, DD), lambda i: (0, 0)),
            pl.BlockSpec((1, DD), lambda i: (0, 0)),
            pl.BlockSpec((1, DD), lambda i: (0, 0)),
        ],
        out_specs=pl.BlockSpec((RB, DD), lambda i: (i, 0)),
        out_shape=jax.ShapeDtypeStruct((NN, DD), jnp.float32),
    )(y, st, g.reshape(1, DD), b.reshape(1, DD))


def _mlp1_body(x_ref, h1_ref, h2_ref, y3_ref, st3_ref, g2_ref, b2_ref,
               w1_ref, bb1_ref, z_ref, st_ref, acc_ref):
    i = pl.program_id(0)

    @pl.when(i == 0)
    def _init():
        acc_ref[...] = jnp.zeros_like(acc_ref)

    mu3 = st3_ref[0:1, :] / NN
    var3 = st3_ref[1:2, :] / NN - mu3 * mu3
    sc3 = g2_ref[...] * lax.rsqrt(var3 + 1e-5)
    h3 = jnp.maximum((y3_ref[...] - mu3) * sc3 + b2_ref[...], 0.0)
    z = (jnp.dot(x_ref[...], w1_ref[0 * DD:1 * DD], preferred_element_type=jnp.float32)
         + jnp.dot(h1_ref[...], w1_ref[1 * DD:2 * DD], preferred_element_type=jnp.float32)
         + jnp.dot(h2_ref[...], w1_ref[2 * DD:3 * DD], preferred_element_type=jnp.float32)
         + jnp.dot(h3, w1_ref[3 * DD:4 * DD], preferred_element_type=jnp.float32)
         + bb1_ref[...])
    z_ref[...] = z
    acc_ref[0:1, :] += jnp.sum(z, axis=0, keepdims=True)
    acc_ref[1:2, :] += jnp.sum(z * z, axis=0, keepdims=True)

    @pl.when(i == pl.num_programs(0) - 1)
    def _fin():
        st_ref[...] = acc_ref[...]


def _mlp1(x, h1, h2, y3, st3, g2, b2, w1, bb1):
    return pl.pallas_call(
        _mlp1_body,
        grid=(NN // RB,),
        in_specs=[
            pl.BlockSpec((RB, DD), lambda i: (i, 0)),
            pl.BlockSpec((RB, DD), lambda i: (i, 0)),
            pl.BlockSpec((RB, DD), lambda i: (i, 0)),
            pl.BlockSpec((RB, DD), lambda i: (i, 0)),
            pl.BlockSpec((2, DD), lambda i: (0, 0)),
            pl.BlockSpec((1, DD), lambda i: (0, 0)),
            pl.BlockSpec((1, DD), lambda i: (0, 0)),
            pl.BlockSpec((4 * DD, 2 * CC), lambda i: (0, 0)),
            pl.BlockSpec((1, 2 * CC), lambda i: (0, 0)),
        ],
        out_specs=[
            pl.BlockSpec((RB, 2 * CC), lambda i: (i, 0)),
            pl.BlockSpec((2, 2 * CC), lambda i: (0, 0)),
        ],
        out_shape=[
            jax.ShapeDtypeStruct((NN, 2 * CC), jnp.float32),
            jax.ShapeDtypeStruct((2, 2 * CC), jnp.float32),
        ],
        scratch_shapes=[pltpu.VMEM((2, 2 * CC), jnp.float32)],
    )(x, h1, h2, y3, st3, g2.reshape(1, DD), b2.reshape(1, DD), w1,
      bb1.reshape(1, 2 * CC))


def _mlp2_body(z_ref, st_ref, gm_ref, bm_ref, w2_ref, bb2_ref, o_ref):
    mu = st_ref[0:1, :] / NN
    var = st_ref[1:2, :] / NN - mu * mu
    scale = gm_ref[...] * lax.rsqrt(var + 1e-5)
    t = (z_ref[...] - mu) * scale + bm_ref[...]
    t = jnp.where(t >= 0.0, t, 0.01 * t)
    o_ref[...] = jnp.dot(t, w2_ref[...], preferred_element_type=jnp.float32) + bb2_ref[...]


def _mlp2(z, st, gm, bm, w2, bb2):
    return pl.pallas_call(
        _mlp2_body,
        grid=(NN // RB,),
        in_specs=[
            pl.BlockSpec((RB, 2 * CC), lambda i: (i, 0)),
            pl.BlockSpec((2, 2 * CC), lambda i: (0, 0)),
            pl.BlockSpec((1, 2 * CC), lambda i: (0, 0)),
            pl.BlockSpec((1, 2 * CC), lambda i: (0, 0)),
            pl.BlockSpec((2 * CC, CC), lambda i: (0, 0)),
            pl.BlockSpec((1, CC), lambda i: (0, 0)),
        ],
        out_specs=pl.BlockSpec((RB, CC), lambda i: (i, 0)),
        out_shape=jax.ShapeDtypeStruct((NN, CC), jnp.float32),
    )(z, st, gm.reshape(1, 2 * CC), bm.reshape(1, 2 * CC), w2, bb2.reshape(1, CC))


def kernel(x, edge_index, Wl0, Wr0, Wl1, Wr1, Wl2, Wr2,
           g0, b0, g1, b1, g2, b2, W1, bb1, gm, bm, W2, bb2):
    src = edge_index[0]
    dst = edge_index[1]
    pad = EPAD - EE
    pad_src = jnp.arange(pad, dtype=jnp.int32) % NN
    src2d = jnp.concatenate([src, pad_src]).reshape(EPAD // CHUNK, CHUNK)
    pad_dst = NN + (jnp.arange(pad, dtype=jnp.int32) % (NACC - NN))
    dst2d = jnp.concatenate([dst, pad_dst]).reshape(EPAD // CHUNK, CHUNK)
    zstripe = jnp.zeros((STRIPE, DD), jnp.float32)
    ones = jnp.ones((CHUNK, DD), jnp.float32)

    sc_cnt = _get_sc_cnt()
    sc_agg = _get_sc_agg()
    pc = sc_cnt(dst2d, zstripe, ones)
    pf0 = sc_agg(x, src2d, dst2d, zstripe)
    y1, st1 = _conv(x, pf0, pc, Wl0, Wr0)
    h1 = _bnrelu(y1, st1, g0, b0)

    pf1 = sc_agg(h1, src2d, dst2d, zstripe)
    y2, st2 = _conv(h1, pf1, pc, Wl1, Wr1)
    h2 = _bnrelu(y2, st2, g1, b1)

    pf2 = sc_agg(h2, src2d, dst2d, zstripe)
    y3, st3 = _conv(h2, pf2, pc, Wl2, Wr2)

    z, stz = _mlp1(x, h1, h2, y3, st3, g2, b2, W1, bb1)
    return _mlp2(z, stz, gm, bm, W2, bb2)
